# baseline (speedup 1.0000x reference)
import functools

import jax
import jax.numpy as jnp
from jax import lax
from jax.experimental import pallas as pl
from jax.experimental.pallas import tpu as pltpu
from jax.experimental.pallas import tpu_sc as plsc

N_NODES = 10000
N_EDGES = 160000
N_FACES = 20000

BN = 1000
BE = 2000

NC, NS, NL = 2, 16, 16
NW = NC * NS
NPAD = 10240
NSLICE = NPAD // NS
CI = 128
NV = 3 * N_FACES
NVPAD = 61440
CF = 160
EPAD = 163840
SBE = 256


def _sc_mesh():
    return plsc.VectorSubcoreMesh(core_axis_name="c", subcore_axis_name="s",
                                  num_cores=NC, num_subcores=NS)


_SC_PARAMS = pltpu.CompilerParams(needs_layout_passes=False)


def _wid():
    return lax.axis_index("s") * NC + lax.axis_index("c")


def _zero_fill(ref, nwords):
    def z(i, _):
        ref[pl.ds(i * NL, NL)] = jnp.zeros((NL,), jnp.float32)
        return 0
    lax.fori_loop(0, nwords // NL, z, 0)


def _rsqrt(s):
    ib = plsc.bitcast(s, jnp.int32)
    y = plsc.bitcast(jnp.int32(0x5F3759DF) - (ib >> 1), jnp.float32)
    y = y * (1.5 - 0.5 * s * y * y)
    y = y * (1.5 - 0.5 * s * y * y)
    y = y * (1.5 - 0.5 * s * y * y)
    return y


def _fa_body(coord_hbm, f0_hbm, f1_hbm, f2_hbm, fa_hbm, ctab, f0, f1, f2,
             fout):
    w = _wid()
    pltpu.sync_copy(coord_hbm, ctab)
    nch = N_FACES // CF

    def chunk(kc, _):
        cid = w + NW * kc

        @pl.when(cid < nch)
        def _():
            base = cid * CF
            pltpu.sync_copy(f0_hbm.at[pl.ds(base, CF)], f0)
            pltpu.sync_copy(f1_hbm.at[pl.ds(base, CF)], f1)
            pltpu.sync_copy(f2_hbm.at[pl.ds(base, CF)], f2)

            def grp(g, _):
                sl = pl.ds(g * NL, NL)
                i0, i1, i2 = f0[sl], f1[sl], f2[sl]

                def getc(iv, j):
                    return plsc.load_gather(ctab, [iv * 4 + j])

                x0, y0, z0 = getc(i0, 0), getc(i0, 1), getc(i0, 2)
                ax, ay, az = getc(i1, 0) - x0, getc(i1, 1) - y0, getc(i1, 2) - z0
                bx, by, bz = getc(i2, 0) - x0, getc(i2, 1) - y0, getc(i2, 2) - z0
                fx = ay * bz - az * by
                fy = az * bx - ax * bz
                fz = ax * by - ay * bx
                s = fx * fx + fy * fy + fz * fz
                fout[sl] = 0.5 * s * _rsqrt(s)
                return 0

            lax.fori_loop(0, CF // NL, grp, 0)
            pltpu.sync_copy(fout, fa_hbm.at[pl.ds(base, CF)])
        return 0

    lax.fori_loop(0, (nch + NW - 1) // NW, chunk, 0)


def _fa_call(coord, face0, face1, face2):
    return pl.kernel(
        _fa_body,
        out_type=jax.ShapeDtypeStruct((N_FACES,), jnp.float32),
        mesh=_sc_mesh(),
        compiler_params=_SC_PARAMS,
        scratch_types=[
            pltpu.VMEM((NPAD * 4,), jnp.float32),
            pltpu.VMEM((CF,), jnp.int32),
            pltpu.VMEM((CF,), jnp.int32),
            pltpu.VMEM((CF,), jnp.int32),
            pltpu.VMEM((CF,), jnp.float32),
        ],
    )(coord, face0, face1, face2)


def _a2_body(fa_hbm, fidx_hbm, vsrc_hbm, out_hbm, fatab, fi_v, vs_v, vals,
             acc, stage):
    c, s = lax.axis_index("c"), lax.axis_index("s")
    w = _wid()
    pltpu.sync_copy(fa_hbm, fatab)
    _zero_fill(stage, NSLICE)
    pltpu.sync_copy(stage, acc.at[pl.ds(s * NSLICE, NSLICE)])
    plsc.subcore_barrier()
    nch = NVPAD // CI

    def chunk(kc, _):
        cid = w + NW * kc
        base = cid * CI
        pltpu.sync_copy(fidx_hbm.at[pl.ds(base, CI)], fi_v)
        pltpu.sync_copy(vsrc_hbm.at[pl.ds(base, CI)], vs_v)

        def grp(g, _):
            sl = pl.ds(g * NL, NL)
            vals[sl] = plsc.load_gather(fatab, [fi_v[sl]])
            return 0

        lax.fori_loop(0, CI // NL, grp, 0)
        pltpu.sync_copy(vals, acc.at[vs_v], add=True)
        return 0

    lax.fori_loop(0, nch // NW, chunk, 0)
    plsc.subcore_barrier()
    pltpu.sync_copy(acc.at[pl.ds(s * NSLICE, NSLICE)], stage)
    pltpu.sync_copy(stage, out_hbm.at[pl.ds(c * NPAD + s * NSLICE, NSLICE)])


def _a2_call(fa, fidx_pad, vsrc_pad):
    return pl.kernel(
        _a2_body,
        out_type=jax.ShapeDtypeStruct((NC * NPAD,), jnp.float32),
        mesh=_sc_mesh(),
        compiler_params=_SC_PARAMS,
        scratch_types=[
            pltpu.VMEM((N_FACES,), jnp.float32),
            pltpu.VMEM((CI,), jnp.int32),
            pltpu.VMEM((CI,), jnp.int32),
            pltpu.VMEM((CI,), jnp.float32),
            pltpu.VMEM_SHARED((NPAD,), jnp.float32),
            pltpu.VMEM((NSLICE,), jnp.float32),
        ],
    )(fa, fidx_pad, vsrc_pad)


def _eg_body(xa_hbm, xb_hbm, coord_hbm, row_hbm, col_hbm, pre_hbm, d4_hbm,
             ctab, row_v, col_v, bufa, bufb, d4buf, sem):
    hid = bufa.shape[1]
    w = _wid()
    pltpu.sync_copy(coord_hbm, ctab)
    iota = lax.iota(jnp.int32, NL)
    base0 = w * (EPAD // NW)

    def sc_loop(ksc, _):
        base = base0 + ksc * SBE
        pltpu.sync_copy(row_hbm.at[pl.ds(base, SBE)], row_v)
        pltpu.sync_copy(col_hbm.at[pl.ds(base, SBE)], col_v)
        descs = []
        for k in range(SBE // CI):
            ks = pl.ds(k * CI, CI)
            descs.append(pltpu.async_copy(
                xa_hbm.at[row_v.at[ks]], bufa.at[ks, :], sem))
            descs.append(pltpu.async_copy(
                xb_hbm.at[col_v.at[ks]], bufb.at[ks, :], sem))

        def grp(g, _):
            sl = pl.ds(g * NL, NL)
            rv, cv = row_v[sl], col_v[sl]

            def diffc(j):
                return (plsc.load_gather(ctab, [rv * 4 + j])
                        - plsc.load_gather(ctab, [cv * 4 + j]))

            dx, dy, dz = diffc(0), diffc(1), diffc(2)
            rad = dx * dx + dy * dy + dz * dz
            fl = g * 64 + iota * 4
            plsc.store_scatter(d4buf, [fl], dx)
            plsc.store_scatter(d4buf, [fl + 1], dy)
            plsc.store_scatter(d4buf, [fl + 2], dz)
            plsc.store_scatter(d4buf, [fl + 3], rad)
            return 0

        lax.fori_loop(0, SBE // NL, grp, 0)
        pltpu.sync_copy(d4buf, d4_hbm.at[pl.ds(base * 4, SBE * 4)])
        for d in descs:
            d.wait()

        def addrow(i, _):
            for j in range(hid // NL):
                sl = pl.ds(j * NL, NL)
                bufa[i, sl] = bufa[i, sl] + bufb[i, sl]
            return 0

        lax.fori_loop(0, SBE, addrow, 0)
        pltpu.sync_copy(bufa, pre_hbm.at[pl.ds(base, SBE), :])
        return 0

    lax.fori_loop(0, EPAD // NW // SBE, sc_loop, 0)


def _eg_call(xa, xb, coord4, row_g, col_g):
    hid = xa.shape[1]
    return pl.kernel(
        _eg_body,
        out_type=[
            jax.ShapeDtypeStruct((EPAD, hid), jnp.float32),
            jax.ShapeDtypeStruct((EPAD * 4,), jnp.float32),
        ],
        mesh=_sc_mesh(),
        compiler_params=_SC_PARAMS,
        scratch_types=[
            pltpu.VMEM((NPAD * 4,), jnp.float32),
            pltpu.VMEM((SBE,), jnp.int32),
            pltpu.VMEM((SBE,), jnp.int32),
            pltpu.VMEM((SBE, hid), jnp.float32),
            pltpu.VMEM((SBE, hid), jnp.float32),
            pltpu.VMEM((SBE * 4,), jnp.float32),
            pltpu.SemaphoreType.DMA,
        ],
    )(xa, xb, coord4, row_g, col_g)


MHALF = NPAD // 2
MROWS = MHALF + NL
MSL = MHALF // NS
SBS = 256


def _es_body(mij_hbm, tw_hbm, d4_hbm, row_hbm, coord_hbm, inv_hbm,
             magg_hbm, cnew_hbm,
             mbuf, row_v, idxm, d4c, twc, dtw, idxj, macc, cacc, stage2,
             ctst, cast, invst, sem):
    hid = mbuf.shape[1]
    c, s = lax.axis_index("c"), lax.axis_index("s")
    iota = lax.iota(jnp.int32, NL)

    def zrow(i, _):
        for j in range(hid // NL):
            stage2[i, pl.ds(j * NL, NL)] = jnp.zeros((NL,), jnp.float32)
        return 0

    lax.fori_loop(0, MSL, zrow, 0)
    pltpu.sync_copy(stage2, macc.at[pl.ds(s * MSL, MSL), :])

    @pl.when(s == 0)
    def _():
        pltpu.sync_copy(stage2.at[pl.ds(0, NL), :],
                        macc.at[pl.ds(MHALF, NL), :])

    _zero_fill(cast, MSL * 4)
    pltpu.sync_copy(cast.at[pl.ds(0, MSL * 4)],
                    cacc.at[pl.ds(s * MSL * 4, MSL * 4)])

    @pl.when(s == 0)
    def _():
        pltpu.sync_copy(cast.at[pl.ds(0, NL)],
                        cacc.at[pl.ds(MHALF * 4, NL)])

    plsc.subcore_barrier()

    nbase = c * MHALF
    base0 = s * (EPAD // NS)

    def sc_loop(ksc, _):
        base = base0 + ksc * SBS
        pltpu.sync_copy(row_hbm.at[pl.ds(base, SBS)], row_v)
        pltpu.sync_copy(mij_hbm.at[pl.ds(base, SBS), :], mbuf)

        for k in range(SBS // CI):
            def remap(gg, _, k=k):
                sl = pl.ds(k * CI + gg * NL, NL)
                loc = row_v[sl] - nbase
                ok = (loc >= 0) & (loc < MHALF)
                idxm[k, pl.ds(gg * NL, NL)] = jnp.where(ok, loc, MHALF)
                return 0

            lax.fori_loop(0, CI // NL, remap, 0)
        descs = []
        for k in range(SBS // CI):
            descs.append(pltpu.async_copy(
                mbuf.at[pl.ds(k * CI, CI), :], macc.at[idxm.at[k]], sem,
                add=True))

        pltpu.sync_copy(d4_hbm.at[pl.ds(base * 4, SBS * 4)], d4c)
        pltpu.sync_copy(tw_hbm.at[pl.ds(base, SBS)], twc)
        for k in range(SBS // CI):
            for j in range(3):
                r = k * 3 + j

                def grp(g, _, k=k, j=j, r=r):
                    sl16 = pl.ds(k * CI + g * NL, NL)
                    e16 = k * CI + g * NL + iota
                    dv = plsc.load_gather(d4c, [e16 * 4 + j])
                    dtw[r, pl.ds(g * NL, NL)] = dv * twc[sl16]
                    loc = row_v[sl16] - nbase
                    ok = (loc >= 0) & (loc < MHALF)
                    idxj[r, pl.ds(g * NL, NL)] = jnp.where(
                        ok, loc * 4 + j, MHALF * 4)
                    return 0

                lax.fori_loop(0, CI // NL, grp, 0)
        descs2 = []
        for r in range(6):
            descs2.append(pltpu.async_copy(dtw.at[r], cacc.at[idxj.at[r]],
                                           sem, add=True))
        for d in descs + descs2:
            d.wait()
        return 0

    lax.fori_loop(0, EPAD // NS // SBS, sc_loop, 0)
    plsc.subcore_barrier()

    pltpu.sync_copy(macc.at[pl.ds(s * MSL, MSL), :], stage2)
    pltpu.sync_copy(stage2, magg_hbm.at[pl.ds(nbase + s * MSL, MSL), :])

    gbase = nbase + s * MSL
    fsl = pl.ds(gbase * 4, MSL * 4)
    pltpu.sync_copy(coord_hbm.at[fsl], ctst)
    pltpu.sync_copy(cacc.at[pl.ds(s * MSL * 4, MSL * 4)], cast)
    pltpu.sync_copy(inv_hbm.at[pl.ds(gbase, MSL)], invst)

    def out_grp(g, _):
        sl = pl.ds(g * NL, NL)
        iv = plsc.load_gather(invst, [g * 4 + (iota >> 2)])
        cast[sl] = ctst[sl] + cast[sl] * iv
        return 0

    lax.fori_loop(0, MSL * 4 // NL, out_grp, 0)
    pltpu.sync_copy(cast.at[pl.ds(0, MSL * 4)], cnew_hbm.at[fsl])


def _es_call(mij, tw_flat, d4, row_s, coord4, inv_row_flat):
    hid = mij.shape[1]
    return pl.kernel(
        _es_body,
        out_type=[
            jax.ShapeDtypeStruct((NPAD, hid), jnp.float32),
            jax.ShapeDtypeStruct((NPAD * 4,), jnp.float32),
        ],
        mesh=_sc_mesh(),
        compiler_params=_SC_PARAMS,
        scratch_types=[
            pltpu.VMEM((SBS, hid), jnp.float32),
            pltpu.VMEM((SBS,), jnp.int32),
            pltpu.VMEM((SBS // CI, CI), jnp.int32),
            pltpu.VMEM((SBS * 4,), jnp.float32),
            pltpu.VMEM((SBS,), jnp.float32),
            pltpu.VMEM((6, CI), jnp.float32),
            pltpu.VMEM((6, CI), jnp.int32),
            pltpu.VMEM_SHARED((MROWS, hid), jnp.float32),
            pltpu.VMEM_SHARED((MHALF * 4 + NL,), jnp.float32),
            pltpu.VMEM((MSL, hid), jnp.float32),
            pltpu.VMEM((MSL * 4,), jnp.float32),
            pltpu.VMEM((MSL * 4,), jnp.float32),
            pltpu.VMEM((MSL,), jnp.float32),
            pltpu.SemaphoreType.DMA,
        ],
    )(mij, tw_flat, d4, row_s, coord4, inv_row_flat)


def _cnt_body(row_hbm, vsrc_hbm, out_hbm, ones_v, idx_v, accr, accv, stage):
    c, s = lax.axis_index("c"), lax.axis_index("s")
    w = _wid()

    def o(i, _):
        ones_v[pl.ds(i * NL, NL)] = jnp.ones((NL,), jnp.float32)
        return 0

    lax.fori_loop(0, CI // NL, o, 0)
    _zero_fill(stage, NSLICE)
    pltpu.sync_copy(stage, accr.at[pl.ds(s * NSLICE, NSLICE)])
    pltpu.sync_copy(stage, accv.at[pl.ds(s * NSLICE, NSLICE)])
    plsc.subcore_barrier()

    nch_r = N_EDGES // CI

    def chunk_r(kc, _):
        cid = w + NW * kc

        @pl.when(cid < nch_r)
        def _():
            pltpu.sync_copy(row_hbm.at[pl.ds(cid * CI, CI)], idx_v)
            pltpu.sync_copy(ones_v, accr.at[idx_v], add=True)
        return 0

    lax.fori_loop(0, (nch_r + NW - 1) // NW, chunk_r, 0)

    nch_v = NVPAD // CI

    def chunk_v(kc, _):
        cid = w + NW * kc
        pltpu.sync_copy(vsrc_hbm.at[pl.ds(cid * CI, CI)], idx_v)
        pltpu.sync_copy(ones_v, accv.at[idx_v], add=True)
        return 0

    lax.fori_loop(0, nch_v // NW, chunk_v, 0)
    plsc.subcore_barrier()
    sl = pl.ds(s * NSLICE, NSLICE)
    pltpu.sync_copy(accr.at[sl], stage)
    pltpu.sync_copy(stage, out_hbm.at[pl.ds((c * 2 + 0) * NPAD + s * NSLICE,
                                            NSLICE)])
    pltpu.sync_copy(accv.at[sl], stage)
    pltpu.sync_copy(stage, out_hbm.at[pl.ds((c * 2 + 1) * NPAD + s * NSLICE,
                                            NSLICE)])


def _cnt_call(row, vsrc_pad):
    return pl.kernel(
        _cnt_body,
        out_type=jax.ShapeDtypeStruct((NC * 2 * NPAD,), jnp.float32),
        mesh=_sc_mesh(),
        compiler_params=_SC_PARAMS,
        scratch_types=[
            pltpu.VMEM((CI,), jnp.float32),
            pltpu.VMEM((CI,), jnp.int32),
            pltpu.VMEM_SHARED((NPAD,), jnp.float32),
            pltpu.VMEM_SHARED((NPAD,), jnp.float32),
            pltpu.VMEM((NSLICE,), jnp.float32),
        ],
    )(row, vsrc_pad)


def _silu(x):
    return x * jax.nn.sigmoid(x)


def _feat_body(area_ref, hks_ref, wf_ref, bf_ref, wa_ref, wb_ref,
               x_ref, xa_ref, xb_ref):
    area = area_ref[...]
    hks = hks_ref[...]
    x = area * wf_ref[0:1, :] + jnp.dot(
        hks, wf_ref[1:, :], preferred_element_type=jnp.float32) + bf_ref[0:1, :]
    x_ref[...] = x
    xa_ref[...] = jnp.dot(x, wa_ref[...], preferred_element_type=jnp.float32)
    xb_ref[...] = jnp.dot(x, wb_ref[...], preferred_element_type=jnp.float32)


def _feat_call(area, hks, wf, bf, wa, wb):
    n = area.shape[0]
    w0 = wf.shape[1]
    hid = wa.shape[1]
    grid = n // BN
    return pl.pallas_call(
        _feat_body,
        grid=(grid,),
        in_specs=[
            pl.BlockSpec((BN, 1), lambda i: (i, 0)),
            pl.BlockSpec((BN, 9), lambda i: (i, 0)),
            pl.BlockSpec(wf.shape, lambda i: (0, 0)),
            pl.BlockSpec((1, w0), lambda i: (0, 0)),
            pl.BlockSpec(wa.shape, lambda i: (0, 0)),
            pl.BlockSpec(wb.shape, lambda i: (0, 0)),
        ],
        out_specs=[
            pl.BlockSpec((BN, w0), lambda i: (i, 0)),
            pl.BlockSpec((BN, hid), lambda i: (i, 0)),
            pl.BlockSpec((BN, hid), lambda i: (i, 0)),
        ],
        out_shape=[
            jax.ShapeDtypeStruct((n, w0), jnp.float32),
            jax.ShapeDtypeStruct((n, hid), jnp.float32),
            jax.ShapeDtypeStruct((n, hid), jnp.float32),
        ],
    )(area, hks, wf, bf, wa, wb)


def _edge_body(pre_ref, d4_ref, wgt_ref, di_ref,
               wrwd_ref, b1_ref, w2_ref, b2_ref, wx1_ref, bx1_ref,
               wx2_ref, bx2_ref, mij_ref, tw_ref):
    hid = w2_ref.shape[0]
    pre = (pre_ref[...][:, :hid]
           + d4_ref[:, 3:4] * wrwd_ref[0:1, :]
           + wgt_ref[...] * wrwd_ref[1:2, :]
           + di_ref[...] * wrwd_ref[2:3, :]
           + b1_ref[0:1, :])
    m1 = _silu(pre)
    mij = _silu(jnp.dot(m1, w2_ref[...], preferred_element_type=jnp.float32)
                + b2_ref[0:1, :])
    t = _silu(jnp.dot(mij, wx1_ref[...], preferred_element_type=jnp.float32)
              + bx1_ref[0:1, :])
    if hid < 128:
        mij_ref[...] = jnp.concatenate(
            [mij, jnp.zeros((mij.shape[0], 128 - hid), jnp.float32)], axis=1)
    else:
        mij_ref[...] = mij
    tw_ref[...] = (jnp.dot(t, wx2_ref[...], preferred_element_type=jnp.float32)
                   + bx2_ref[0:1, :])


def _edge_call(pre, d4r, weight, di, wrwd, b1, w2, b2, wx1, bx1, wx2, bx2):
    e = pre.shape[0]
    hid = w2.shape[0]
    grid = e // BE
    return pl.pallas_call(
        _edge_body,
        grid=(grid,),
        in_specs=[
            pl.BlockSpec((BE, 128), lambda i: (i, 0)),
            pl.BlockSpec((BE, 4), lambda i: (i, 0)),
            pl.BlockSpec((BE, 1), lambda i: (i, 0)),
            pl.BlockSpec((BE, 1), lambda i: (i, 0)),
            pl.BlockSpec((3, hid), lambda i: (0, 0)),
            pl.BlockSpec((1, hid), lambda i: (0, 0)),
            pl.BlockSpec((hid, hid), lambda i: (0, 0)),
            pl.BlockSpec((1, hid), lambda i: (0, 0)),
            pl.BlockSpec((hid, hid), lambda i: (0, 0)),
            pl.BlockSpec((1, hid), lambda i: (0, 0)),
            pl.BlockSpec((hid, 1), lambda i: (0, 0)),
            pl.BlockSpec((1, 1), lambda i: (0, 0)),
        ],
        out_specs=[
            pl.BlockSpec((BE, 128), lambda i: (i, 0)),
            pl.BlockSpec((BE, 1), lambda i: (i, 0)),
        ],
        out_shape=[
            jax.ShapeDtypeStruct((EPAD, 128), jnp.float32),
            jax.ShapeDtypeStruct((EPAD, 1), jnp.float32),
        ],
    )(pre, d4r, weight, di, wrwd, b1, w2, b2, wx1, bx1, wx2, bx2)


def _node_body(is_last, x_ref, magg_ref, area_ref, w1x_ref, w1m_ref, w1a_ref,
               b1_ref, w2_ref, b2_ref, wa_ref, wb_ref, wa2_ref, wb2_ref,
               xo_ref, xa_ref, xb_ref):
    hid = w1m_ref.shape[0]
    magg = magg_ref[...][:, :hid]
    h = _silu(jnp.dot(x_ref[...], w1x_ref[...],
                      preferred_element_type=jnp.float32)
              + jnp.dot(magg, w1m_ref[...],
                        preferred_element_type=jnp.float32)
              + area_ref[...] * w1a_ref[0:1, :]
              + b1_ref[0:1, :])
    xo = jnp.dot(h, w2_ref[...], preferred_element_type=jnp.float32) + b2_ref[0:1, :]
    if is_last:
        h2 = jnp.maximum(
            jnp.dot(xo, wa_ref[...], preferred_element_type=jnp.float32)
            + wa2_ref[0:1, :], 0.0)
        lg = (jnp.dot(h2, wb_ref[...], preferred_element_type=jnp.float32)
              + wb2_ref[0:1, :])
        mx = jnp.max(lg, axis=1, keepdims=True)
        s = lg - mx
        lse = jnp.log(jnp.sum(jnp.exp(s), axis=1, keepdims=True))
        xo_ref[...] = s - lse
        xa_ref[...] = jnp.zeros_like(xa_ref)
        xb_ref[...] = jnp.zeros_like(xb_ref)
    else:
        xo_ref[...] = xo
        xa_ref[...] = jnp.dot(xo, wa_ref[...], preferred_element_type=jnp.float32)
        xb_ref[...] = jnp.dot(xo, wb_ref[...], preferred_element_type=jnp.float32)


def _node_call(is_last, x, magg, area, w1x, w1m, w1a, b1, w2, b2,
               wa, wb, wa2, wb2, out_dim, hid_next):
    n = x.shape[0]
    fi = x.shape[1]
    grid = n // BN
    full = lambda a: pl.BlockSpec(a.shape, lambda i: (0,) * a.ndim)
    return pl.pallas_call(
        functools.partial(_node_body, is_last),
        grid=(grid,),
        in_specs=[
            pl.BlockSpec((BN, fi), lambda i: (i, 0)),
            pl.BlockSpec((BN, 128), lambda i: (i, 0)),
            pl.BlockSpec((BN, 1), lambda i: (i, 0)),
            full(w1x), full(w1m), full(w1a), full(b1), full(w2), full(b2),
            full(wa), full(wb), full(wa2), full(wb2),
        ],
        out_specs=[
            pl.BlockSpec((BN, out_dim), lambda i: (i, 0)),
            pl.BlockSpec((BN, hid_next), lambda i: (i, 0)),
            pl.BlockSpec((BN, hid_next), lambda i: (i, 0)),
        ],
        out_shape=[
            jax.ShapeDtypeStruct((n, out_dim), jnp.float32),
            jax.ShapeDtypeStruct((n, hid_next), jnp.float32),
            jax.ShapeDtypeStruct((n, hid_next), jnp.float32),
        ],
    )(x, magg, area, w1x, w1m, w1a, b1, w2, b2, wa, wb, wa2, wb2)


def _seg_sum(vals, ids, n):
    return jax.ops.segment_sum(vals, ids, num_segments=n)


def _coord2area(face, coord):
    v1 = coord[face[1]] - coord[face[0]]
    v2 = coord[face[2]] - coord[face[0]]
    fn = jnp.cross(v1, v2)
    return jnp.linalg.norm(fn, axis=-1) / 2.0


def kernel(pos, hks, weight, di_angles, params, edge_index, face, vertex2face):
    n = pos.shape[0]
    row, col = edge_index[0], edge_index[1]
    vsrc, fidx = vertex2face[:, 0], vertex2face[:, 1]
    vsrc_pad = jnp.concatenate(
        [vsrc.astype(jnp.int32),
         jnp.full((NVPAD - NV,), N_NODES, jnp.int32)])
    fidx_pad = jnp.concatenate(
        [fidx.astype(jnp.int32), jnp.zeros((NVPAD - NV,), jnp.int32)])
    row32 = row.astype(jnp.int32)
    row_g = jnp.pad(row32, (0, EPAD - N_EDGES))
    col_g = jnp.pad(col.astype(jnp.int32), (0, EPAD - N_EDGES))
    row_s = jnp.pad(row32, (0, EPAD - N_EDGES), constant_values=NPAD)

    p = pos - jnp.mean(pos, axis=0)
    m = jnp.max(jnp.sqrt(jnp.sum(p ** 2, axis=1)))
    coord4 = jnp.pad(p / m, ((0, NPAD - n), (0, 1))).reshape(-1)

    cnt = _cnt_call(row32, vsrc_pad).reshape(NC, 2, NPAD)
    inv_row_flat = 1.0 / jnp.clip(cnt[0, 0] + cnt[1, 0], 1.0)
    inv_v2f = 1.0 / jnp.clip(cnt[0, 1, :n] + cnt[1, 1, :n], 1.0)[:, None]
    face0 = face[0].astype(jnp.int32)
    face1 = face[1].astype(jnp.int32)
    face2 = face[2].astype(jnp.int32)

    def area_of(c4):
        fa = _fa_call(c4, face0, face1, face2)
        asum = _a2_call(fa, fidx_pad, vsrc_pad).reshape(NC, NPAD)
        return (asum[0, :n] + asum[1, :n])[:, None] * inv_v2f

    area = area_of(coord4)

    wf, bf = params['feat']
    dims = [(32, 64, 32), (64, 128, 64), (128, 256, 128)]

    def e1_split(i, fi):
        w, b = params['c%d_e1' % i]
        hid = w.shape[1]
        wa = jnp.pad(w[:fi], ((0, 0), (0, 128 - hid)))
        wb = jnp.pad(w[fi:2 * fi], ((0, 0), (0, 128 - hid)))
        return wa, wb, w[2 * fi:2 * fi + 3], b[None, :]

    wa0, wb0, wrwd0, b1e0 = e1_split(0, 32)
    x, xa, xb = _feat_call(area, hks, wf, bf[None, :], wa0, wb0)

    for i, (fi, fo, hid) in enumerate(dims):
        _, _, wrwd, b1e = e1_split(i, fi)
        w2, b2 = params['c%d_e2' % i]
        wx1, bx1 = params['c%d_x1' % i]
        wx2, bx2 = params['c%d_x2' % i]
        wn1, bn1 = params['c%d_n1' % i]
        wn2, bn2 = params['c%d_n2' % i]
        w1x, w1m, w1a = wn1[:fi], wn1[fi:fi + hid], wn1[fi + hid:fi + hid + 1]

        pre, d4 = _eg_call(xa, xb, coord4, row_g, col_g)
        mij, tw = _edge_call(pre, d4.reshape(EPAD, 4), weight,
                             di_angles, wrwd, b1e, w2, b2[None, :],
                             wx1, bx1[None, :], wx2, bx2[None, :])
        magg, coord4 = _es_call(mij, tw.reshape(-1), d4, row_s, coord4,
                                inv_row_flat)
        area_i = area_of(coord4)

        if i < 2:
            fi2 = dims[i + 1][0]
            wa, wb, _, _ = e1_split(i + 1, fi2)
            hid_next = 128
            x, xa, xb = _node_call(
                False, x, magg, area_i, w1x, w1m, w1a, bn1[None, :],
                wn2, bn2[None, :], wa, wb,
                jnp.zeros((1, hid_next), jnp.float32),
                jnp.zeros((1, hid_next), jnp.float32), fo, hid_next)
        else:
            wl1, bl1 = params['lin1']
            wl2, bl2 = params['lin2']
            x, _, _ = _node_call(
                True, x, magg, area_i, w1x, w1m, w1a, bn1[None, :],
                wn2, bn2[None, :], wl1, wl2,
                bl1[None, :], bl2[None, :], 16, 8)
    return x


# speedup vs baseline: 1.1780x; 1.1780x over previous
import functools

import jax
import jax.numpy as jnp
from jax import lax
from jax.experimental import pallas as pl
from jax.experimental.pallas import tpu as pltpu
from jax.experimental.pallas import tpu_sc as plsc

N_NODES = 10000
N_EDGES = 160000
N_FACES = 20000

BN = 1000
BE = 2000

NC, NS, NL = 2, 16, 16
NW = NC * NS
NPAD = 10240
NSLICE = NPAD // NS
CI = 128
NV = 3 * N_FACES
NVPAD = 61440
CF = 160
EPAD = 163840
SBE = 256


def _sc_mesh():
    return plsc.VectorSubcoreMesh(core_axis_name="c", subcore_axis_name="s",
                                  num_cores=NC, num_subcores=NS)


_SC_PARAMS = pltpu.CompilerParams(needs_layout_passes=False)


def _wid():
    return lax.axis_index("s") * NC + lax.axis_index("c")


def _zero_fill(ref, nwords):
    def z(i, _):
        ref[pl.ds(i * NL, NL)] = jnp.zeros((NL,), jnp.float32)
        return 0
    lax.fori_loop(0, nwords // NL, z, 0)


def _rsqrt(s):
    ib = plsc.bitcast(s, jnp.int32)
    y = plsc.bitcast(jnp.int32(0x5F3759DF) - (ib >> 1), jnp.float32)
    y = y * (1.5 - 0.5 * s * y * y)
    y = y * (1.5 - 0.5 * s * y * y)
    y = y * (1.5 - 0.5 * s * y * y)
    return y


def _fa_body(coord_hbm, f0_hbm, f1_hbm, f2_hbm, fa_hbm, ctab, f0, f1, f2,
             fout):
    w = _wid()
    pltpu.sync_copy(coord_hbm, ctab)
    nch = N_FACES // CF

    def chunk(kc, _):
        cid = w + NW * kc

        @pl.when(cid < nch)
        def _():
            base = cid * CF
            pltpu.sync_copy(f0_hbm.at[pl.ds(base, CF)], f0)
            pltpu.sync_copy(f1_hbm.at[pl.ds(base, CF)], f1)
            pltpu.sync_copy(f2_hbm.at[pl.ds(base, CF)], f2)

            def grp(g, _):
                sl = pl.ds(g * NL, NL)
                i0, i1, i2 = f0[sl], f1[sl], f2[sl]

                def getc(iv, j):
                    return plsc.load_gather(ctab, [iv * 4 + j])

                x0, y0, z0 = getc(i0, 0), getc(i0, 1), getc(i0, 2)
                ax, ay, az = getc(i1, 0) - x0, getc(i1, 1) - y0, getc(i1, 2) - z0
                bx, by, bz = getc(i2, 0) - x0, getc(i2, 1) - y0, getc(i2, 2) - z0
                fx = ay * bz - az * by
                fy = az * bx - ax * bz
                fz = ax * by - ay * bx
                s = fx * fx + fy * fy + fz * fz
                fout[sl] = 0.5 * s * _rsqrt(s)
                return 0

            lax.fori_loop(0, CF // NL, grp, 0)
            pltpu.sync_copy(fout, fa_hbm.at[pl.ds(base, CF)])
        return 0

    lax.fori_loop(0, (nch + NW - 1) // NW, chunk, 0)


def _fa_call(coord, face0, face1, face2):
    return pl.kernel(
        _fa_body,
        out_type=jax.ShapeDtypeStruct((N_FACES,), jnp.float32),
        mesh=_sc_mesh(),
        compiler_params=_SC_PARAMS,
        scratch_types=[
            pltpu.VMEM((NPAD * 4,), jnp.float32),
            pltpu.VMEM((CF,), jnp.int32),
            pltpu.VMEM((CF,), jnp.int32),
            pltpu.VMEM((CF,), jnp.int32),
            pltpu.VMEM((CF,), jnp.float32),
        ],
    )(coord, face0, face1, face2)


def _a2_body(fa_hbm, fidx_hbm, vsrc_hbm, out_hbm, fatab, fi_v, vs_v, vals,
             acc, stage):
    c, s = lax.axis_index("c"), lax.axis_index("s")
    w = _wid()
    pltpu.sync_copy(fa_hbm, fatab)
    _zero_fill(stage, NSLICE)
    pltpu.sync_copy(stage, acc.at[pl.ds(s * NSLICE, NSLICE)])
    plsc.subcore_barrier()
    nch = NVPAD // CI

    def chunk(kc, _):
        cid = w + NW * kc
        base = cid * CI
        pltpu.sync_copy(fidx_hbm.at[pl.ds(base, CI)], fi_v)
        pltpu.sync_copy(vsrc_hbm.at[pl.ds(base, CI)], vs_v)

        def grp(g, _):
            sl = pl.ds(g * NL, NL)
            vals[sl] = plsc.load_gather(fatab, [fi_v[sl]])
            return 0

        lax.fori_loop(0, CI // NL, grp, 0)
        pltpu.sync_copy(vals, acc.at[vs_v], add=True)
        return 0

    lax.fori_loop(0, nch // NW, chunk, 0)
    plsc.subcore_barrier()
    pltpu.sync_copy(acc.at[pl.ds(s * NSLICE, NSLICE)], stage)
    pltpu.sync_copy(stage, out_hbm.at[pl.ds(c * NPAD + s * NSLICE, NSLICE)])


def _a2_call(fa, fidx_pad, vsrc_pad):
    return pl.kernel(
        _a2_body,
        out_type=jax.ShapeDtypeStruct((NC * NPAD,), jnp.float32),
        mesh=_sc_mesh(),
        compiler_params=_SC_PARAMS,
        scratch_types=[
            pltpu.VMEM((N_FACES,), jnp.float32),
            pltpu.VMEM((CI,), jnp.int32),
            pltpu.VMEM((CI,), jnp.int32),
            pltpu.VMEM((CI,), jnp.float32),
            pltpu.VMEM_SHARED((NPAD,), jnp.float32),
            pltpu.VMEM((NSLICE,), jnp.float32),
        ],
    )(fa, fidx_pad, vsrc_pad)


def _eg_body(xa_hbm, xb_hbm, coord_hbm, row_hbm, col_hbm, pre_hbm, d4_hbm,
             ctab, row_v, col_v, bufa, bufb, d4buf, sem):
    hid = bufa.shape[1]
    w = _wid()
    pltpu.sync_copy(coord_hbm, ctab)
    iota = lax.iota(jnp.int32, NL)
    base0 = w * (EPAD // NW)

    def sc_loop(ksc, _):
        base = base0 + ksc * SBE
        pltpu.sync_copy(row_hbm.at[pl.ds(base, SBE)], row_v)
        pltpu.sync_copy(col_hbm.at[pl.ds(base, SBE)], col_v)
        descs = []
        for k in range(SBE // CI):
            ks = pl.ds(k * CI, CI)
            descs.append(pltpu.async_copy(
                xa_hbm.at[row_v.at[ks]], bufa.at[ks, :], sem))
            descs.append(pltpu.async_copy(
                xb_hbm.at[col_v.at[ks]], bufb.at[ks, :], sem))

        def grp(g, _):
            sl = pl.ds(g * NL, NL)
            rv, cv = row_v[sl], col_v[sl]

            def diffc(j):
                return (plsc.load_gather(ctab, [rv * 4 + j])
                        - plsc.load_gather(ctab, [cv * 4 + j]))

            dx, dy, dz = diffc(0), diffc(1), diffc(2)
            rad = dx * dx + dy * dy + dz * dz
            fl = g * 64 + iota * 4
            plsc.store_scatter(d4buf, [fl], dx)
            plsc.store_scatter(d4buf, [fl + 1], dy)
            plsc.store_scatter(d4buf, [fl + 2], dz)
            plsc.store_scatter(d4buf, [fl + 3], rad)
            return 0

        lax.fori_loop(0, SBE // NL, grp, 0)
        pltpu.sync_copy(d4buf, d4_hbm.at[pl.ds(base * 4, SBE * 4)])
        for d in descs:
            d.wait()

        def addrow(i, _):
            for j in range(hid // NL):
                sl = pl.ds(j * NL, NL)
                bufa[i, sl] = bufa[i, sl] + bufb[i, sl]
            return 0

        lax.fori_loop(0, SBE, addrow, 0)
        pltpu.sync_copy(bufa, pre_hbm.at[pl.ds(base, SBE), :])
        return 0

    lax.fori_loop(0, EPAD // NW // SBE, sc_loop, 0)


def _eg_call(xa, xb, coord4, row_g, col_g):
    hid = xa.shape[1]
    return pl.kernel(
        _eg_body,
        out_type=[
            jax.ShapeDtypeStruct((EPAD, hid), jnp.float32),
            jax.ShapeDtypeStruct((EPAD * 4,), jnp.float32),
        ],
        mesh=_sc_mesh(),
        compiler_params=_SC_PARAMS,
        scratch_types=[
            pltpu.VMEM((NPAD * 4,), jnp.float32),
            pltpu.VMEM((SBE,), jnp.int32),
            pltpu.VMEM((SBE,), jnp.int32),
            pltpu.VMEM((SBE, hid), jnp.float32),
            pltpu.VMEM((SBE, hid), jnp.float32),
            pltpu.VMEM((SBE * 4,), jnp.float32),
            pltpu.SemaphoreType.DMA,
        ],
    )(xa, xb, coord4, row_g, col_g)


MHALF = NPAD // 2
MROWS = MHALF + NL
MSL = MHALF // NS
SBS = 256


def _es_body(packed, mij_hbm, tw_hbm, d4_hbm, row_hbm, coord_hbm, inv_hbm,
             magg_hbm, cnew_hbm,
             mbuf, row_v, idxm, d4c, twc, dtw, idxj, macc, cacc, stage2,
             ctst, cast, invst, sem):
    hid = mbuf.shape[1]
    c, s = lax.axis_index("c"), lax.axis_index("s")
    iota = lax.iota(jnp.int32, NL)

    def zrow(i, _):
        for j in range(hid // NL):
            stage2[i, pl.ds(j * NL, NL)] = jnp.zeros((NL,), jnp.float32)
        return 0

    lax.fori_loop(0, MSL, zrow, 0)
    pltpu.sync_copy(stage2, macc.at[pl.ds(s * MSL, MSL), :])

    @pl.when(s == 0)
    def _():
        pltpu.sync_copy(stage2.at[pl.ds(0, NL), :],
                        macc.at[pl.ds(MHALF, NL), :])

    if not packed:
        _zero_fill(cast, NSLICE * 4)

        @pl.when(c == 0)
        def _():
            pltpu.sync_copy(cast, cacc.at[pl.ds(s * NSLICE * 4, NSLICE * 4)])

            @pl.when(s == 0)
            def _():
                pltpu.sync_copy(cast.at[pl.ds(0, NL)],
                                cacc.at[pl.ds(NPAD * 4, NL)])

    plsc.subcore_barrier()

    nbase = c * MHALF
    base0 = s * (EPAD // NS)

    def sc_loop(ksc, _):
        base = base0 + ksc * SBS
        pltpu.sync_copy(row_hbm.at[pl.ds(base, SBS)], row_v)
        pltpu.sync_copy(mij_hbm.at[pl.ds(base, SBS), :], mbuf)
        for k in range(SBS // CI):
            def remap(gg, _, k=k):
                sl = pl.ds(k * CI + gg * NL, NL)
                loc = row_v[sl] - nbase
                ok = (loc >= 0) & (loc < MHALF)
                idxm[k, pl.ds(gg * NL, NL)] = jnp.where(ok, loc, MHALF)
                return 0

            lax.fori_loop(0, CI // NL, remap, 0)
        descs = []
        for k in range(SBS // CI):
            descs.append(pltpu.async_copy(
                mbuf.at[pl.ds(k * CI, CI), :], macc.at[idxm.at[k]], sem,
                add=True))

        if not packed:
            @pl.when(c == 0)
            def _():
                pltpu.sync_copy(d4_hbm.at[pl.ds(base * 4, SBS * 4)], d4c)
                pltpu.sync_copy(tw_hbm.at[pl.ds(base, SBS)], twc)
                for k in range(SBS // CI):
                    for j in range(3):
                        r = k * 3 + j

                        def grp(g, _, k=k, j=j, r=r):
                            sl16 = pl.ds(k * CI + g * NL, NL)
                            e16 = k * CI + g * NL + iota
                            dv = plsc.load_gather(d4c, [e16 * 4 + j])
                            dtw[r, pl.ds(g * NL, NL)] = dv * twc[sl16]
                            idxj[r, pl.ds(g * NL, NL)] = row_v[sl16] * 4 + j
                            return 0

                        lax.fori_loop(0, CI // NL, grp, 0)
                descs2 = []
                for r in range(6):
                    descs2.append(pltpu.async_copy(
                        dtw.at[r], cacc.at[idxj.at[r]], sem, add=True))
                for d in descs2:
                    d.wait()

        for d in descs:
            d.wait()
        return 0

    lax.fori_loop(0, EPAD // NS // SBS, sc_loop, 0)
    plsc.subcore_barrier()

    pltpu.sync_copy(macc.at[pl.ds(s * MSL, MSL), :], stage2)
    pltpu.sync_copy(stage2, magg_hbm.at[pl.ds(nbase + s * MSL, MSL), :])

    if packed:
        gbase = nbase + s * MSL
        fsl = pl.ds(gbase * 4, MSL * 4)
        pltpu.sync_copy(coord_hbm.at[fsl], ctst.at[pl.ds(0, MSL * 4)])
        pltpu.sync_copy(inv_hbm.at[pl.ds(gbase, MSL)], invst.at[pl.ds(0, MSL)])

        def extract(i, _):
            x = stage2[i, pl.ds(112, NL)]
            idx = jnp.where(iota >= 12, i * 4 + iota - 12, 0)
            plsc.store_scatter(cast, [idx], x, mask=iota >= 12)
            return 0

        lax.fori_loop(0, MSL, extract, 0)

        def out_grp(g, _):
            sl = pl.ds(g * NL, NL)
            iv = plsc.load_gather(invst, [g * 4 + (iota >> 2)])
            cast[sl] = ctst[sl] + cast[sl] * iv
            return 0

        lax.fori_loop(0, MSL * 4 // NL, out_grp, 0)
        pltpu.sync_copy(cast.at[pl.ds(0, MSL * 4)], cnew_hbm.at[fsl])
    else:
        @pl.when(c == 0)
        def _():
            fsl = pl.ds(s * NSLICE * 4, NSLICE * 4)
            rsl = pl.ds(s * NSLICE, NSLICE)
            pltpu.sync_copy(coord_hbm.at[fsl], ctst)
            pltpu.sync_copy(cacc.at[fsl], cast)
            pltpu.sync_copy(inv_hbm.at[rsl], invst)

            def out_grp(g, _):
                sl = pl.ds(g * NL, NL)
                iv = plsc.load_gather(invst, [g * 4 + (iota >> 2)])
                cast[sl] = ctst[sl] + cast[sl] * iv
                return 0

            lax.fori_loop(0, NSLICE * 4 // NL, out_grp, 0)
            pltpu.sync_copy(cast, cnew_hbm.at[fsl])


def _es_call(packed, mij, tw_flat, d4, row_s, coord4, inv_row_flat):
    hid = mij.shape[1]
    return pl.kernel(
        functools.partial(_es_body, packed),
        out_type=[
            jax.ShapeDtypeStruct((NPAD, hid), jnp.float32),
            jax.ShapeDtypeStruct((NPAD * 4,), jnp.float32),
        ],
        mesh=_sc_mesh(),
        compiler_params=_SC_PARAMS,
        scratch_types=[
            pltpu.VMEM((SBS, hid), jnp.float32),
            pltpu.VMEM((SBS,), jnp.int32),
            pltpu.VMEM((SBS // CI, CI), jnp.int32),
            pltpu.VMEM((SBS * 4,), jnp.float32),
            pltpu.VMEM((SBS,), jnp.float32),
            pltpu.VMEM((6, CI), jnp.float32),
            pltpu.VMEM((6, CI), jnp.int32),
            pltpu.VMEM_SHARED((MROWS, hid), jnp.float32),
            pltpu.VMEM_SHARED((NPAD * 4 + NL,), jnp.float32),
            pltpu.VMEM((MSL, hid), jnp.float32),
            pltpu.VMEM((NSLICE * 4,), jnp.float32),
            pltpu.VMEM((NSLICE * 4,), jnp.float32),
            pltpu.VMEM((NSLICE,), jnp.float32),
            pltpu.SemaphoreType.DMA,
        ],
    )(mij, tw_flat, d4, row_s, coord4, inv_row_flat)


def _cnt_body(row_hbm, vsrc_hbm, out_hbm, ones_v, idx_v, accr, accv, stage):
    c, s = lax.axis_index("c"), lax.axis_index("s")
    w = _wid()

    def o(i, _):
        ones_v[pl.ds(i * NL, NL)] = jnp.ones((NL,), jnp.float32)
        return 0

    lax.fori_loop(0, CI // NL, o, 0)
    _zero_fill(stage, NSLICE)
    pltpu.sync_copy(stage, accr.at[pl.ds(s * NSLICE, NSLICE)])
    pltpu.sync_copy(stage, accv.at[pl.ds(s * NSLICE, NSLICE)])
    plsc.subcore_barrier()

    nch_r = N_EDGES // CI

    def chunk_r(kc, _):
        cid = w + NW * kc

        @pl.when(cid < nch_r)
        def _():
            pltpu.sync_copy(row_hbm.at[pl.ds(cid * CI, CI)], idx_v)
            pltpu.sync_copy(ones_v, accr.at[idx_v], add=True)
        return 0

    lax.fori_loop(0, (nch_r + NW - 1) // NW, chunk_r, 0)

    nch_v = NVPAD // CI

    def chunk_v(kc, _):
        cid = w + NW * kc
        pltpu.sync_copy(vsrc_hbm.at[pl.ds(cid * CI, CI)], idx_v)
        pltpu.sync_copy(ones_v, accv.at[idx_v], add=True)
        return 0

    lax.fori_loop(0, nch_v // NW, chunk_v, 0)
    plsc.subcore_barrier()
    sl = pl.ds(s * NSLICE, NSLICE)
    pltpu.sync_copy(accr.at[sl], stage)
    pltpu.sync_copy(stage, out_hbm.at[pl.ds((c * 2 + 0) * NPAD + s * NSLICE,
                                            NSLICE)])
    pltpu.sync_copy(accv.at[sl], stage)
    pltpu.sync_copy(stage, out_hbm.at[pl.ds((c * 2 + 1) * NPAD + s * NSLICE,
                                            NSLICE)])


def _cnt_call(row, vsrc_pad):
    return pl.kernel(
        _cnt_body,
        out_type=jax.ShapeDtypeStruct((NC * 2 * NPAD,), jnp.float32),
        mesh=_sc_mesh(),
        compiler_params=_SC_PARAMS,
        scratch_types=[
            pltpu.VMEM((CI,), jnp.float32),
            pltpu.VMEM((CI,), jnp.int32),
            pltpu.VMEM_SHARED((NPAD,), jnp.float32),
            pltpu.VMEM_SHARED((NPAD,), jnp.float32),
            pltpu.VMEM((NSLICE,), jnp.float32),
        ],
    )(row, vsrc_pad)


def _silu(x):
    return x * jax.nn.sigmoid(x)


def _feat_body(area_ref, hks_ref, wf_ref, bf_ref, wa_ref, wb_ref,
               x_ref, xa_ref, xb_ref):
    area = area_ref[...]
    hks = hks_ref[...]
    x = area * wf_ref[0:1, :] + jnp.dot(
        hks, wf_ref[1:, :], preferred_element_type=jnp.float32) + bf_ref[0:1, :]
    x_ref[...] = x
    xa_ref[...] = jnp.dot(x, wa_ref[...], preferred_element_type=jnp.float32)
    xb_ref[...] = jnp.dot(x, wb_ref[...], preferred_element_type=jnp.float32)


def _feat_call(area, hks, wf, bf, wa, wb):
    n = area.shape[0]
    w0 = wf.shape[1]
    hid = wa.shape[1]
    grid = n // BN
    return pl.pallas_call(
        _feat_body,
        grid=(grid,),
        in_specs=[
            pl.BlockSpec((BN, 1), lambda i: (i, 0)),
            pl.BlockSpec((BN, 9), lambda i: (i, 0)),
            pl.BlockSpec(wf.shape, lambda i: (0, 0)),
            pl.BlockSpec((1, w0), lambda i: (0, 0)),
            pl.BlockSpec(wa.shape, lambda i: (0, 0)),
            pl.BlockSpec(wb.shape, lambda i: (0, 0)),
        ],
        out_specs=[
            pl.BlockSpec((BN, w0), lambda i: (i, 0)),
            pl.BlockSpec((BN, hid), lambda i: (i, 0)),
            pl.BlockSpec((BN, hid), lambda i: (i, 0)),
        ],
        out_shape=[
            jax.ShapeDtypeStruct((n, w0), jnp.float32),
            jax.ShapeDtypeStruct((n, hid), jnp.float32),
            jax.ShapeDtypeStruct((n, hid), jnp.float32),
        ],
    )(area, hks, wf, bf, wa, wb)


def _edge_body(pre_ref, d4_ref, wgt_ref, di_ref,
               wrwd_ref, b1_ref, w2_ref, b2_ref, wx1_ref, bx1_ref,
               wx2_ref, bx2_ref, mij_ref, tw_ref):
    hid = w2_ref.shape[0]
    pre = (pre_ref[...][:, :hid]
           + d4_ref[:, 3:4] * wrwd_ref[0:1, :]
           + wgt_ref[...] * wrwd_ref[1:2, :]
           + di_ref[...] * wrwd_ref[2:3, :]
           + b1_ref[0:1, :])
    m1 = _silu(pre)
    mij = _silu(jnp.dot(m1, w2_ref[...], preferred_element_type=jnp.float32)
                + b2_ref[0:1, :])
    t = _silu(jnp.dot(mij, wx1_ref[...], preferred_element_type=jnp.float32)
              + bx1_ref[0:1, :])
    tw = (jnp.dot(t, wx2_ref[...], preferred_element_type=jnp.float32)
          + bx2_ref[0:1, :])
    if hid <= 124:
        dtw = d4_ref[...] * tw
        mij_ref[...] = jnp.concatenate(
            [mij, jnp.zeros((mij.shape[0], 124 - hid), jnp.float32), dtw],
            axis=1)
    else:
        mij_ref[...] = mij
    tw_ref[...] = tw


def _edge_call(pre, d4r, weight, di, wrwd, b1, w2, b2, wx1, bx1, wx2, bx2):
    e = pre.shape[0]
    hid = w2.shape[0]
    grid = e // BE
    return pl.pallas_call(
        _edge_body,
        grid=(grid,),
        in_specs=[
            pl.BlockSpec((BE, 128), lambda i: (i, 0)),
            pl.BlockSpec((BE, 4), lambda i: (i, 0)),
            pl.BlockSpec((BE, 1), lambda i: (i, 0)),
            pl.BlockSpec((BE, 1), lambda i: (i, 0)),
            pl.BlockSpec((3, hid), lambda i: (0, 0)),
            pl.BlockSpec((1, hid), lambda i: (0, 0)),
            pl.BlockSpec((hid, hid), lambda i: (0, 0)),
            pl.BlockSpec((1, hid), lambda i: (0, 0)),
            pl.BlockSpec((hid, hid), lambda i: (0, 0)),
            pl.BlockSpec((1, hid), lambda i: (0, 0)),
            pl.BlockSpec((hid, 1), lambda i: (0, 0)),
            pl.BlockSpec((1, 1), lambda i: (0, 0)),
        ],
        out_specs=[
            pl.BlockSpec((BE, 128), lambda i: (i, 0)),
            pl.BlockSpec((BE, 1), lambda i: (i, 0)),
        ],
        out_shape=[
            jax.ShapeDtypeStruct((EPAD, 128), jnp.float32),
            jax.ShapeDtypeStruct((EPAD, 1), jnp.float32),
        ],
    )(pre, d4r, weight, di, wrwd, b1, w2, b2, wx1, bx1, wx2, bx2)


def _node_body(is_last, x_ref, magg_ref, area_ref, w1x_ref, w1m_ref, w1a_ref,
               b1_ref, w2_ref, b2_ref, wa_ref, wb_ref, wa2_ref, wb2_ref,
               xo_ref, xa_ref, xb_ref):
    hid = w1m_ref.shape[0]
    magg = magg_ref[...][:, :hid]
    h = _silu(jnp.dot(x_ref[...], w1x_ref[...],
                      preferred_element_type=jnp.float32)
              + jnp.dot(magg, w1m_ref[...],
                        preferred_element_type=jnp.float32)
              + area_ref[...] * w1a_ref[0:1, :]
              + b1_ref[0:1, :])
    xo = jnp.dot(h, w2_ref[...], preferred_element_type=jnp.float32) + b2_ref[0:1, :]
    if is_last:
        h2 = jnp.maximum(
            jnp.dot(xo, wa_ref[...], preferred_element_type=jnp.float32)
            + wa2_ref[0:1, :], 0.0)
        lg = (jnp.dot(h2, wb_ref[...], preferred_element_type=jnp.float32)
              + wb2_ref[0:1, :])
        mx = jnp.max(lg, axis=1, keepdims=True)
        s = lg - mx
        lse = jnp.log(jnp.sum(jnp.exp(s), axis=1, keepdims=True))
        xo_ref[...] = s - lse
        xa_ref[...] = jnp.zeros_like(xa_ref)
        xb_ref[...] = jnp.zeros_like(xb_ref)
    else:
        xo_ref[...] = xo
        xa_ref[...] = jnp.dot(xo, wa_ref[...], preferred_element_type=jnp.float32)
        xb_ref[...] = jnp.dot(xo, wb_ref[...], preferred_element_type=jnp.float32)


def _node_call(is_last, x, magg, area, w1x, w1m, w1a, b1, w2, b2,
               wa, wb, wa2, wb2, out_dim, hid_next):
    n = x.shape[0]
    fi = x.shape[1]
    grid = n // BN
    full = lambda a: pl.BlockSpec(a.shape, lambda i: (0,) * a.ndim)
    return pl.pallas_call(
        functools.partial(_node_body, is_last),
        grid=(grid,),
        in_specs=[
            pl.BlockSpec((BN, fi), lambda i: (i, 0)),
            pl.BlockSpec((BN, 128), lambda i: (i, 0)),
            pl.BlockSpec((BN, 1), lambda i: (i, 0)),
            full(w1x), full(w1m), full(w1a), full(b1), full(w2), full(b2),
            full(wa), full(wb), full(wa2), full(wb2),
        ],
        out_specs=[
            pl.BlockSpec((BN, out_dim), lambda i: (i, 0)),
            pl.BlockSpec((BN, hid_next), lambda i: (i, 0)),
            pl.BlockSpec((BN, hid_next), lambda i: (i, 0)),
        ],
        out_shape=[
            jax.ShapeDtypeStruct((n, out_dim), jnp.float32),
            jax.ShapeDtypeStruct((n, hid_next), jnp.float32),
            jax.ShapeDtypeStruct((n, hid_next), jnp.float32),
        ],
    )(x, magg, area, w1x, w1m, w1a, b1, w2, b2, wa, wb, wa2, wb2)


def _seg_sum(vals, ids, n):
    return jax.ops.segment_sum(vals, ids, num_segments=n)


def _coord2area(face, coord):
    v1 = coord[face[1]] - coord[face[0]]
    v2 = coord[face[2]] - coord[face[0]]
    fn = jnp.cross(v1, v2)
    return jnp.linalg.norm(fn, axis=-1) / 2.0


def kernel(pos, hks, weight, di_angles, params, edge_index, face, vertex2face):
    n = pos.shape[0]
    row, col = edge_index[0], edge_index[1]
    vsrc, fidx = vertex2face[:, 0], vertex2face[:, 1]
    vsrc_pad = jnp.concatenate(
        [vsrc.astype(jnp.int32),
         jnp.full((NVPAD - NV,), N_NODES, jnp.int32)])
    fidx_pad = jnp.concatenate(
        [fidx.astype(jnp.int32), jnp.zeros((NVPAD - NV,), jnp.int32)])
    row32 = row.astype(jnp.int32)
    row_g = jnp.pad(row32, (0, EPAD - N_EDGES))
    col_g = jnp.pad(col.astype(jnp.int32), (0, EPAD - N_EDGES))
    row_s = jnp.pad(row32, (0, EPAD - N_EDGES), constant_values=NPAD)

    p = pos - jnp.mean(pos, axis=0)
    m = jnp.max(jnp.sqrt(jnp.sum(p ** 2, axis=1)))
    coord4 = jnp.pad(p / m, ((0, NPAD - n), (0, 1))).reshape(-1)

    cnt = _cnt_call(row32, vsrc_pad).reshape(NC, 2, NPAD)
    inv_row_flat = 1.0 / jnp.clip(cnt[0, 0] + cnt[1, 0], 1.0)
    inv_v2f = 1.0 / jnp.clip(cnt[0, 1, :n] + cnt[1, 1, :n], 1.0)[:, None]
    face0 = face[0].astype(jnp.int32)
    face1 = face[1].astype(jnp.int32)
    face2 = face[2].astype(jnp.int32)

    def area_of(c4):
        fa = _fa_call(c4, face0, face1, face2)
        asum = _a2_call(fa, fidx_pad, vsrc_pad).reshape(NC, NPAD)
        return (asum[0, :n] + asum[1, :n])[:, None] * inv_v2f

    area = area_of(coord4)

    wf, bf = params['feat']
    dims = [(32, 64, 32), (64, 128, 64), (128, 256, 128)]

    def e1_split(i, fi):
        w, b = params['c%d_e1' % i]
        hid = w.shape[1]
        wa = jnp.pad(w[:fi], ((0, 0), (0, 128 - hid)))
        wb = jnp.pad(w[fi:2 * fi], ((0, 0), (0, 128 - hid)))
        return wa, wb, w[2 * fi:2 * fi + 3], b[None, :]

    wa0, wb0, wrwd0, b1e0 = e1_split(0, 32)
    x, xa, xb = _feat_call(area, hks, wf, bf[None, :], wa0, wb0)

    for i, (fi, fo, hid) in enumerate(dims):
        _, _, wrwd, b1e = e1_split(i, fi)
        w2, b2 = params['c%d_e2' % i]
        wx1, bx1 = params['c%d_x1' % i]
        wx2, bx2 = params['c%d_x2' % i]
        wn1, bn1 = params['c%d_n1' % i]
        wn2, bn2 = params['c%d_n2' % i]
        w1x, w1m, w1a = wn1[:fi], wn1[fi:fi + hid], wn1[fi + hid:fi + hid + 1]

        pre, d4 = _eg_call(xa, xb, coord4, row_g, col_g)
        mij, tw = _edge_call(pre, d4.reshape(EPAD, 4), weight,
                             di_angles, wrwd, b1e, w2, b2[None, :],
                             wx1, bx1[None, :], wx2, bx2[None, :])
        magg, coord4 = _es_call(hid <= 124, mij, tw.reshape(-1), d4, row_s,
                                coord4, inv_row_flat)
        area_i = area_of(coord4)

        if i < 2:
            fi2 = dims[i + 1][0]
            wa, wb, _, _ = e1_split(i + 1, fi2)
            hid_next = 128
            x, xa, xb = _node_call(
                False, x, magg, area_i, w1x, w1m, w1a, bn1[None, :],
                wn2, bn2[None, :], wa, wb,
                jnp.zeros((1, hid_next), jnp.float32),
                jnp.zeros((1, hid_next), jnp.float32), fo, hid_next)
        else:
            wl1, bl1 = params['lin1']
            wl2, bl2 = params['lin2']
            x, _, _ = _node_call(
                True, x, magg, area_i, w1x, w1m, w1a, bn1[None, :],
                wn2, bn2[None, :], wl1, wl2,
                bl1[None, :], bl2[None, :], 16, 8)
    return x


# speedup vs baseline: 1.1978x; 1.0168x over previous
import functools

import jax
import jax.numpy as jnp
from jax import lax
from jax.experimental import pallas as pl
from jax.experimental.pallas import tpu as pltpu
from jax.experimental.pallas import tpu_sc as plsc

N_NODES = 10000
N_EDGES = 160000
N_FACES = 20000

BN = 1000
BE = 2000

NC, NS, NL = 2, 16, 16
NW = NC * NS
NPAD = 10240
NSLICE = NPAD // NS
CI = 128
NV = 3 * N_FACES
NVPAD = 61440
CF = 160
EPAD = 163840
SBE = 256


def _sc_mesh():
    return plsc.VectorSubcoreMesh(core_axis_name="c", subcore_axis_name="s",
                                  num_cores=NC, num_subcores=NS)


_SC_PARAMS = pltpu.CompilerParams(needs_layout_passes=False)


def _wid():
    return lax.axis_index("s") * NC + lax.axis_index("c")


def _zero_fill(ref, nwords):
    def z(i, _):
        ref[pl.ds(i * NL, NL)] = jnp.zeros((NL,), jnp.float32)
        return 0
    lax.fori_loop(0, nwords // NL, z, 0)


def _rsqrt(s):
    ib = plsc.bitcast(s, jnp.int32)
    y = plsc.bitcast(jnp.int32(0x5F3759DF) - (ib >> 1), jnp.float32)
    y = y * (1.5 - 0.5 * s * y * y)
    y = y * (1.5 - 0.5 * s * y * y)
    y = y * (1.5 - 0.5 * s * y * y)
    return y


def _fa_body(coord_hbm, f0_hbm, f1_hbm, f2_hbm, fa_hbm, ctab, f0, f1, f2,
             fout):
    w = _wid()
    pltpu.sync_copy(coord_hbm, ctab)
    nch = N_FACES // CF

    def chunk(kc, _):
        cid = w + NW * kc

        @pl.when(cid < nch)
        def _():
            base = cid * CF
            pltpu.sync_copy(f0_hbm.at[pl.ds(base, CF)], f0)
            pltpu.sync_copy(f1_hbm.at[pl.ds(base, CF)], f1)
            pltpu.sync_copy(f2_hbm.at[pl.ds(base, CF)], f2)

            def grp(g, _):
                sl = pl.ds(g * NL, NL)
                i0, i1, i2 = f0[sl], f1[sl], f2[sl]

                def getc(iv, j):
                    return plsc.load_gather(ctab, [iv * 4 + j])

                x0, y0, z0 = getc(i0, 0), getc(i0, 1), getc(i0, 2)
                ax, ay, az = getc(i1, 0) - x0, getc(i1, 1) - y0, getc(i1, 2) - z0
                bx, by, bz = getc(i2, 0) - x0, getc(i2, 1) - y0, getc(i2, 2) - z0
                fx = ay * bz - az * by
                fy = az * bx - ax * bz
                fz = ax * by - ay * bx
                s = fx * fx + fy * fy + fz * fz
                fout[sl] = 0.5 * s * _rsqrt(s)
                return 0

            lax.fori_loop(0, CF // NL, grp, 0)
            pltpu.sync_copy(fout, fa_hbm.at[pl.ds(base, CF)])
        return 0

    lax.fori_loop(0, (nch + NW - 1) // NW, chunk, 0)


def _fa_call(coord, face0, face1, face2):
    return pl.kernel(
        _fa_body,
        out_type=jax.ShapeDtypeStruct((N_FACES,), jnp.float32),
        mesh=_sc_mesh(),
        compiler_params=_SC_PARAMS,
        scratch_types=[
            pltpu.VMEM((NPAD * 4,), jnp.float32),
            pltpu.VMEM((CF,), jnp.int32),
            pltpu.VMEM((CF,), jnp.int32),
            pltpu.VMEM((CF,), jnp.int32),
            pltpu.VMEM((CF,), jnp.float32),
        ],
    )(coord, face0, face1, face2)


def _a2_body(fa_hbm, fidx_hbm, vsrc_hbm, out_hbm, fatab, fi_v, vs_v, vals,
             acc, stage):
    c, s = lax.axis_index("c"), lax.axis_index("s")
    w = _wid()
    pltpu.sync_copy(fa_hbm, fatab)
    _zero_fill(stage, NSLICE)
    pltpu.sync_copy(stage, acc.at[pl.ds(s * NSLICE, NSLICE)])
    plsc.subcore_barrier()
    nch = NVPAD // CI

    def chunk(kc, _):
        cid = w + NW * kc
        base = cid * CI
        pltpu.sync_copy(fidx_hbm.at[pl.ds(base, CI)], fi_v)
        pltpu.sync_copy(vsrc_hbm.at[pl.ds(base, CI)], vs_v)

        def grp(g, _):
            sl = pl.ds(g * NL, NL)
            vals[sl] = plsc.load_gather(fatab, [fi_v[sl]])
            return 0

        lax.fori_loop(0, CI // NL, grp, 0)
        pltpu.sync_copy(vals, acc.at[vs_v], add=True)
        return 0

    lax.fori_loop(0, nch // NW, chunk, 0)
    plsc.subcore_barrier()
    pltpu.sync_copy(acc.at[pl.ds(s * NSLICE, NSLICE)], stage)
    pltpu.sync_copy(stage, out_hbm.at[pl.ds(c * NPAD + s * NSLICE, NSLICE)])


def _a2_call(fa, fidx_pad, vsrc_pad):
    return pl.kernel(
        _a2_body,
        out_type=jax.ShapeDtypeStruct((NC * NPAD,), jnp.float32),
        mesh=_sc_mesh(),
        compiler_params=_SC_PARAMS,
        scratch_types=[
            pltpu.VMEM((N_FACES,), jnp.float32),
            pltpu.VMEM((CI,), jnp.int32),
            pltpu.VMEM((CI,), jnp.int32),
            pltpu.VMEM((CI,), jnp.float32),
            pltpu.VMEM_SHARED((NPAD,), jnp.float32),
            pltpu.VMEM((NSLICE,), jnp.float32),
        ],
    )(fa, fidx_pad, vsrc_pad)


def _eg_body(xa_hbm, xb_hbm, coord_hbm, row_hbm, col_hbm, pre_hbm, d4_hbm,
             ctab, row_v, col_v, bufa, bufb, d4buf, sem):
    hid = bufa.shape[1]
    w = _wid()
    pltpu.sync_copy(coord_hbm, ctab)
    iota = lax.iota(jnp.int32, NL)
    base0 = w * (EPAD // NW)

    def sc_loop(ksc, _):
        base = base0 + ksc * SBE
        pltpu.sync_copy(row_hbm.at[pl.ds(base, SBE)], row_v)
        pltpu.sync_copy(col_hbm.at[pl.ds(base, SBE)], col_v)
        descs = []
        for k in range(SBE // CI):
            ks = pl.ds(k * CI, CI)
            descs.append(pltpu.async_copy(
                xa_hbm.at[row_v.at[ks]], bufa.at[ks, :], sem))
            descs.append(pltpu.async_copy(
                xb_hbm.at[col_v.at[ks]], bufb.at[ks, :], sem))

        def grp(g, _):
            sl = pl.ds(g * NL, NL)
            rv, cv = row_v[sl], col_v[sl]

            def diffc(j):
                return (plsc.load_gather(ctab, [rv * 4 + j])
                        - plsc.load_gather(ctab, [cv * 4 + j]))

            dx, dy, dz = diffc(0), diffc(1), diffc(2)
            rad = dx * dx + dy * dy + dz * dz
            fl = g * 64 + iota * 4
            plsc.store_scatter(d4buf, [fl], dx)
            plsc.store_scatter(d4buf, [fl + 1], dy)
            plsc.store_scatter(d4buf, [fl + 2], dz)
            plsc.store_scatter(d4buf, [fl + 3], rad)
            return 0

        lax.fori_loop(0, SBE // NL, grp, 0)
        pltpu.sync_copy(d4buf, d4_hbm.at[pl.ds(base * 4, SBE * 4)])
        for d in descs:
            d.wait()

        def addrow(i, _):
            for j in range(hid // NL):
                sl = pl.ds(j * NL, NL)
                bufa[i, sl] = bufa[i, sl] + bufb[i, sl]
            return 0

        lax.fori_loop(0, SBE, addrow, 0)
        pltpu.sync_copy(bufa, pre_hbm.at[pl.ds(base, SBE), :])
        return 0

    lax.fori_loop(0, EPAD // NW // SBE, sc_loop, 0)


def _eg_call(xa, xb, coord4, row_g, col_g):
    hid = xa.shape[1]
    return pl.kernel(
        _eg_body,
        out_type=[
            jax.ShapeDtypeStruct((EPAD, hid), jnp.float32),
            jax.ShapeDtypeStruct((EPAD * 4,), jnp.float32),
        ],
        mesh=_sc_mesh(),
        compiler_params=_SC_PARAMS,
        scratch_types=[
            pltpu.VMEM((NPAD * 4,), jnp.float32),
            pltpu.VMEM((SBE,), jnp.int32),
            pltpu.VMEM((SBE,), jnp.int32),
            pltpu.VMEM((SBE, hid), jnp.float32),
            pltpu.VMEM((SBE, hid), jnp.float32),
            pltpu.VMEM((SBE * 4,), jnp.float32),
            pltpu.SemaphoreType.DMA,
        ],
    )(xa, xb, coord4, row_g, col_g)


MHALF = NPAD // 2
MROWS = MHALF + NL
MSL = MHALF // NS
SBS = 256


def _es_body(packed, mij_hbm, tw_hbm, d4_hbm, row_hbm, coord_hbm, inv_hbm,
             magg_hbm, cnew_hbm,
             mbuf, row_v, idxm, d4c, twc, dtw, idxj, macc, cacc, stage2,
             ctst, cast, invst, sem):
    hid = mbuf.shape[1]
    c, s = lax.axis_index("c"), lax.axis_index("s")
    iota = lax.iota(jnp.int32, NL)

    def zrow(i, _):
        for j in range(hid // NL):
            stage2[i, pl.ds(j * NL, NL)] = jnp.zeros((NL,), jnp.float32)
        return 0

    lax.fori_loop(0, MSL, zrow, 0)
    pltpu.sync_copy(stage2, macc.at[pl.ds(s * MSL, MSL), :])

    @pl.when(s == 0)
    def _():
        pltpu.sync_copy(stage2.at[pl.ds(0, NL), :],
                        macc.at[pl.ds(MHALF, NL), :])

    if not packed:
        _zero_fill(cast, NSLICE * 4)

        @pl.when(c == 0)
        def _():
            pltpu.sync_copy(cast, cacc.at[pl.ds(s * NSLICE * 4, NSLICE * 4)])

            @pl.when(s == 0)
            def _():
                pltpu.sync_copy(cast.at[pl.ds(0, NL)],
                                cacc.at[pl.ds(NPAD * 4, NL)])

    plsc.subcore_barrier()

    nbase = c * MHALF
    base0 = s * (EPAD // NS)

    def sc_loop(ksc, _):
        base = base0 + ksc * SBS
        pltpu.sync_copy(row_hbm.at[pl.ds(base, SBS)], row_v)
        pltpu.sync_copy(mij_hbm.at[pl.ds(base, SBS), :], mbuf)
        for k in range(SBS // CI):
            def remap(gg, _, k=k):
                sl = pl.ds(k * CI + gg * NL, NL)
                loc = row_v[sl] - nbase
                ok = (loc >= 0) & (loc < MHALF)
                idxm[k, pl.ds(gg * NL, NL)] = jnp.where(ok, loc, MHALF + iota)
                return 0

            lax.fori_loop(0, CI // NL, remap, 0)
        descs = []
        for k in range(SBS // CI):
            descs.append(pltpu.async_copy(
                mbuf.at[pl.ds(k * CI, CI), :], macc.at[idxm.at[k]], sem,
                add=True))

        if not packed:
            @pl.when(c == 0)
            def _():
                pltpu.sync_copy(d4_hbm.at[pl.ds(base * 4, SBS * 4)], d4c)
                pltpu.sync_copy(tw_hbm.at[pl.ds(base, SBS)], twc)
                for k in range(SBS // CI):
                    for j in range(3):
                        r = k * 3 + j

                        def grp(g, _, k=k, j=j, r=r):
                            sl16 = pl.ds(k * CI + g * NL, NL)
                            e16 = k * CI + g * NL + iota
                            dv = plsc.load_gather(d4c, [e16 * 4 + j])
                            dtw[r, pl.ds(g * NL, NL)] = dv * twc[sl16]
                            idxj[r, pl.ds(g * NL, NL)] = row_v[sl16] * 4 + j
                            return 0

                        lax.fori_loop(0, CI // NL, grp, 0)
                descs2 = []
                for r in range(6):
                    descs2.append(pltpu.async_copy(
                        dtw.at[r], cacc.at[idxj.at[r]], sem, add=True))
                for d in descs2:
                    d.wait()

        for d in descs:
            d.wait()
        return 0

    lax.fori_loop(0, EPAD // NS // SBS, sc_loop, 0)
    plsc.subcore_barrier()

    pltpu.sync_copy(macc.at[pl.ds(s * MSL, MSL), :], stage2)
    pltpu.sync_copy(stage2, magg_hbm.at[pl.ds(nbase + s * MSL, MSL), :])

    if packed:
        gbase = nbase + s * MSL
        fsl = pl.ds(gbase * 4, MSL * 4)
        pltpu.sync_copy(coord_hbm.at[fsl], ctst.at[pl.ds(0, MSL * 4)])
        pltpu.sync_copy(inv_hbm.at[pl.ds(gbase, MSL)], invst.at[pl.ds(0, MSL)])

        def extract(i, _):
            x = stage2[i, pl.ds(112, NL)]
            idx = jnp.where(iota >= 12, i * 4 + iota - 12, 0)
            plsc.store_scatter(cast, [idx], x, mask=iota >= 12)
            return 0

        lax.fori_loop(0, MSL, extract, 0)

        def out_grp(g, _):
            sl = pl.ds(g * NL, NL)
            iv = plsc.load_gather(invst, [g * 4 + (iota >> 2)])
            cast[sl] = ctst[sl] + cast[sl] * iv
            return 0

        lax.fori_loop(0, MSL * 4 // NL, out_grp, 0)
        pltpu.sync_copy(cast.at[pl.ds(0, MSL * 4)], cnew_hbm.at[fsl])
    else:
        @pl.when(c == 0)
        def _():
            fsl = pl.ds(s * NSLICE * 4, NSLICE * 4)
            rsl = pl.ds(s * NSLICE, NSLICE)
            pltpu.sync_copy(coord_hbm.at[fsl], ctst)
            pltpu.sync_copy(cacc.at[fsl], cast)
            pltpu.sync_copy(inv_hbm.at[rsl], invst)

            def out_grp(g, _):
                sl = pl.ds(g * NL, NL)
                iv = plsc.load_gather(invst, [g * 4 + (iota >> 2)])
                cast[sl] = ctst[sl] + cast[sl] * iv
                return 0

            lax.fori_loop(0, NSLICE * 4 // NL, out_grp, 0)
            pltpu.sync_copy(cast, cnew_hbm.at[fsl])


def _es_call(packed, mij, tw_flat, d4, row_s, coord4, inv_row_flat):
    hid = mij.shape[1]
    return pl.kernel(
        functools.partial(_es_body, packed),
        out_type=[
            jax.ShapeDtypeStruct((NPAD, hid), jnp.float32),
            jax.ShapeDtypeStruct((NPAD * 4,), jnp.float32),
        ],
        mesh=_sc_mesh(),
        compiler_params=_SC_PARAMS,
        scratch_types=[
            pltpu.VMEM((SBS, hid), jnp.float32),
            pltpu.VMEM((SBS,), jnp.int32),
            pltpu.VMEM((SBS // CI, CI), jnp.int32),
            pltpu.VMEM((SBS * 4,), jnp.float32),
            pltpu.VMEM((SBS,), jnp.float32),
            pltpu.VMEM((6, CI), jnp.float32),
            pltpu.VMEM((6, CI), jnp.int32),
            pltpu.VMEM_SHARED((MROWS, hid), jnp.float32),
            pltpu.VMEM_SHARED((NPAD * 4 + NL,), jnp.float32),
            pltpu.VMEM((MSL, hid), jnp.float32),
            pltpu.VMEM((NSLICE * 4,), jnp.float32),
            pltpu.VMEM((NSLICE * 4,), jnp.float32),
            pltpu.VMEM((NSLICE,), jnp.float32),
            pltpu.SemaphoreType.DMA,
        ],
    )(mij, tw_flat, d4, row_s, coord4, inv_row_flat)


def _cnt_body(row_hbm, vsrc_hbm, out_hbm, ones_v, idx_v, accr, accv, stage):
    c, s = lax.axis_index("c"), lax.axis_index("s")
    w = _wid()

    def o(i, _):
        ones_v[pl.ds(i * NL, NL)] = jnp.ones((NL,), jnp.float32)
        return 0

    lax.fori_loop(0, CI // NL, o, 0)
    _zero_fill(stage, NSLICE)
    pltpu.sync_copy(stage, accr.at[pl.ds(s * NSLICE, NSLICE)])
    pltpu.sync_copy(stage, accv.at[pl.ds(s * NSLICE, NSLICE)])
    plsc.subcore_barrier()

    nch_r = N_EDGES // CI

    def chunk_r(kc, _):
        cid = w + NW * kc

        @pl.when(cid < nch_r)
        def _():
            pltpu.sync_copy(row_hbm.at[pl.ds(cid * CI, CI)], idx_v)
            pltpu.sync_copy(ones_v, accr.at[idx_v], add=True)
        return 0

    lax.fori_loop(0, (nch_r + NW - 1) // NW, chunk_r, 0)

    nch_v = NVPAD // CI

    def chunk_v(kc, _):
        cid = w + NW * kc
        pltpu.sync_copy(vsrc_hbm.at[pl.ds(cid * CI, CI)], idx_v)
        pltpu.sync_copy(ones_v, accv.at[idx_v], add=True)
        return 0

    lax.fori_loop(0, nch_v // NW, chunk_v, 0)
    plsc.subcore_barrier()
    sl = pl.ds(s * NSLICE, NSLICE)
    pltpu.sync_copy(accr.at[sl], stage)
    pltpu.sync_copy(stage, out_hbm.at[pl.ds((c * 2 + 0) * NPAD + s * NSLICE,
                                            NSLICE)])
    pltpu.sync_copy(accv.at[sl], stage)
    pltpu.sync_copy(stage, out_hbm.at[pl.ds((c * 2 + 1) * NPAD + s * NSLICE,
                                            NSLICE)])


def _cnt_call(row, vsrc_pad):
    return pl.kernel(
        _cnt_body,
        out_type=jax.ShapeDtypeStruct((NC * 2 * NPAD,), jnp.float32),
        mesh=_sc_mesh(),
        compiler_params=_SC_PARAMS,
        scratch_types=[
            pltpu.VMEM((CI,), jnp.float32),
            pltpu.VMEM((CI,), jnp.int32),
            pltpu.VMEM_SHARED((NPAD,), jnp.float32),
            pltpu.VMEM_SHARED((NPAD,), jnp.float32),
            pltpu.VMEM((NSLICE,), jnp.float32),
        ],
    )(row, vsrc_pad)


def _silu(x):
    return x * jax.nn.sigmoid(x)


def _feat_body(area_ref, hks_ref, wf_ref, bf_ref, wa_ref, wb_ref,
               x_ref, xa_ref, xb_ref):
    area = area_ref[...]
    hks = hks_ref[...]
    x = area * wf_ref[0:1, :] + jnp.dot(
        hks, wf_ref[1:, :], preferred_element_type=jnp.float32) + bf_ref[0:1, :]
    x_ref[...] = x
    xa_ref[...] = jnp.dot(x, wa_ref[...], preferred_element_type=jnp.float32)
    xb_ref[...] = jnp.dot(x, wb_ref[...], preferred_element_type=jnp.float32)


def _feat_call(area, hks, wf, bf, wa, wb):
    n = area.shape[0]
    w0 = wf.shape[1]
    hid = wa.shape[1]
    grid = n // BN
    return pl.pallas_call(
        _feat_body,
        grid=(grid,),
        in_specs=[
            pl.BlockSpec((BN, 1), lambda i: (i, 0)),
            pl.BlockSpec((BN, 9), lambda i: (i, 0)),
            pl.BlockSpec(wf.shape, lambda i: (0, 0)),
            pl.BlockSpec((1, w0), lambda i: (0, 0)),
            pl.BlockSpec(wa.shape, lambda i: (0, 0)),
            pl.BlockSpec(wb.shape, lambda i: (0, 0)),
        ],
        out_specs=[
            pl.BlockSpec((BN, w0), lambda i: (i, 0)),
            pl.BlockSpec((BN, hid), lambda i: (i, 0)),
            pl.BlockSpec((BN, hid), lambda i: (i, 0)),
        ],
        out_shape=[
            jax.ShapeDtypeStruct((n, w0), jnp.float32),
            jax.ShapeDtypeStruct((n, hid), jnp.float32),
            jax.ShapeDtypeStruct((n, hid), jnp.float32),
        ],
    )(area, hks, wf, bf, wa, wb)


def _edge_body(pre_ref, d4_ref, wgt_ref, di_ref,
               wrwd_ref, b1_ref, w2_ref, b2_ref, wx1_ref, bx1_ref,
               wx2_ref, bx2_ref, mij_ref, tw_ref):
    hid = w2_ref.shape[0]
    pre = (pre_ref[...][:, :hid]
           + d4_ref[:, 3:4] * wrwd_ref[0:1, :]
           + wgt_ref[...] * wrwd_ref[1:2, :]
           + di_ref[...] * wrwd_ref[2:3, :]
           + b1_ref[0:1, :])
    m1 = _silu(pre)
    mij = _silu(jnp.dot(m1, w2_ref[...], preferred_element_type=jnp.float32)
                + b2_ref[0:1, :])
    t = _silu(jnp.dot(mij, wx1_ref[...], preferred_element_type=jnp.float32)
              + bx1_ref[0:1, :])
    tw = (jnp.dot(t, wx2_ref[...], preferred_element_type=jnp.float32)
          + bx2_ref[0:1, :])
    if hid <= 124:
        dtw = d4_ref[...] * tw
        mij_ref[...] = jnp.concatenate(
            [mij, jnp.zeros((mij.shape[0], 124 - hid), jnp.float32), dtw],
            axis=1)
    else:
        mij_ref[...] = mij
    tw_ref[...] = tw


def _edge_call(pre, d4r, weight, di, wrwd, b1, w2, b2, wx1, bx1, wx2, bx2):
    e = pre.shape[0]
    hid = w2.shape[0]
    grid = e // BE
    return pl.pallas_call(
        _edge_body,
        grid=(grid,),
        in_specs=[
            pl.BlockSpec((BE, 128), lambda i: (i, 0)),
            pl.BlockSpec((BE, 4), lambda i: (i, 0)),
            pl.BlockSpec((BE, 1), lambda i: (i, 0)),
            pl.BlockSpec((BE, 1), lambda i: (i, 0)),
            pl.BlockSpec((3, hid), lambda i: (0, 0)),
            pl.BlockSpec((1, hid), lambda i: (0, 0)),
            pl.BlockSpec((hid, hid), lambda i: (0, 0)),
            pl.BlockSpec((1, hid), lambda i: (0, 0)),
            pl.BlockSpec((hid, hid), lambda i: (0, 0)),
            pl.BlockSpec((1, hid), lambda i: (0, 0)),
            pl.BlockSpec((hid, 1), lambda i: (0, 0)),
            pl.BlockSpec((1, 1), lambda i: (0, 0)),
        ],
        out_specs=[
            pl.BlockSpec((BE, 128), lambda i: (i, 0)),
            pl.BlockSpec((BE, 1), lambda i: (i, 0)),
        ],
        out_shape=[
            jax.ShapeDtypeStruct((EPAD, 128), jnp.float32),
            jax.ShapeDtypeStruct((EPAD, 1), jnp.float32),
        ],
    )(pre, d4r, weight, di, wrwd, b1, w2, b2, wx1, bx1, wx2, bx2)


def _node_body(is_last, x_ref, magg_ref, area_ref, w1x_ref, w1m_ref, w1a_ref,
               b1_ref, w2_ref, b2_ref, wa_ref, wb_ref, wa2_ref, wb2_ref,
               xo_ref, xa_ref, xb_ref):
    hid = w1m_ref.shape[0]
    magg = magg_ref[...][:, :hid]
    h = _silu(jnp.dot(x_ref[...], w1x_ref[...],
                      preferred_element_type=jnp.float32)
              + jnp.dot(magg, w1m_ref[...],
                        preferred_element_type=jnp.float32)
              + area_ref[...] * w1a_ref[0:1, :]
              + b1_ref[0:1, :])
    xo = jnp.dot(h, w2_ref[...], preferred_element_type=jnp.float32) + b2_ref[0:1, :]
    if is_last:
        h2 = jnp.maximum(
            jnp.dot(xo, wa_ref[...], preferred_element_type=jnp.float32)
            + wa2_ref[0:1, :], 0.0)
        lg = (jnp.dot(h2, wb_ref[...], preferred_element_type=jnp.float32)
              + wb2_ref[0:1, :])
        mx = jnp.max(lg, axis=1, keepdims=True)
        s = lg - mx
        lse = jnp.log(jnp.sum(jnp.exp(s), axis=1, keepdims=True))
        xo_ref[...] = s - lse
        xa_ref[...] = jnp.zeros_like(xa_ref)
        xb_ref[...] = jnp.zeros_like(xb_ref)
    else:
        xo_ref[...] = xo
        xa_ref[...] = jnp.dot(xo, wa_ref[...], preferred_element_type=jnp.float32)
        xb_ref[...] = jnp.dot(xo, wb_ref[...], preferred_element_type=jnp.float32)


def _node_call(is_last, x, magg, area, w1x, w1m, w1a, b1, w2, b2,
               wa, wb, wa2, wb2, out_dim, hid_next):
    n = x.shape[0]
    fi = x.shape[1]
    grid = n // BN
    full = lambda a: pl.BlockSpec(a.shape, lambda i: (0,) * a.ndim)
    return pl.pallas_call(
        functools.partial(_node_body, is_last),
        grid=(grid,),
        in_specs=[
            pl.BlockSpec((BN, fi), lambda i: (i, 0)),
            pl.BlockSpec((BN, 128), lambda i: (i, 0)),
            pl.BlockSpec((BN, 1), lambda i: (i, 0)),
            full(w1x), full(w1m), full(w1a), full(b1), full(w2), full(b2),
            full(wa), full(wb), full(wa2), full(wb2),
        ],
        out_specs=[
            pl.BlockSpec((BN, out_dim), lambda i: (i, 0)),
            pl.BlockSpec((BN, hid_next), lambda i: (i, 0)),
            pl.BlockSpec((BN, hid_next), lambda i: (i, 0)),
        ],
        out_shape=[
            jax.ShapeDtypeStruct((n, out_dim), jnp.float32),
            jax.ShapeDtypeStruct((n, hid_next), jnp.float32),
            jax.ShapeDtypeStruct((n, hid_next), jnp.float32),
        ],
    )(x, magg, area, w1x, w1m, w1a, b1, w2, b2, wa, wb, wa2, wb2)


def _seg_sum(vals, ids, n):
    return jax.ops.segment_sum(vals, ids, num_segments=n)


def _coord2area(face, coord):
    v1 = coord[face[1]] - coord[face[0]]
    v2 = coord[face[2]] - coord[face[0]]
    fn = jnp.cross(v1, v2)
    return jnp.linalg.norm(fn, axis=-1) / 2.0


def kernel(pos, hks, weight, di_angles, params, edge_index, face, vertex2face):
    n = pos.shape[0]
    row, col = edge_index[0], edge_index[1]
    vsrc, fidx = vertex2face[:, 0], vertex2face[:, 1]
    vsrc_pad = jnp.concatenate(
        [vsrc.astype(jnp.int32),
         jnp.full((NVPAD - NV,), N_NODES, jnp.int32)])
    fidx_pad = jnp.concatenate(
        [fidx.astype(jnp.int32), jnp.zeros((NVPAD - NV,), jnp.int32)])
    row32 = row.astype(jnp.int32)
    row_g = jnp.pad(row32, (0, EPAD - N_EDGES))
    col_g = jnp.pad(col.astype(jnp.int32), (0, EPAD - N_EDGES))
    row_s = jnp.pad(row32, (0, EPAD - N_EDGES), constant_values=NPAD)

    p = pos - jnp.mean(pos, axis=0)
    m = jnp.max(jnp.sqrt(jnp.sum(p ** 2, axis=1)))
    coord4 = jnp.pad(p / m, ((0, NPAD - n), (0, 1))).reshape(-1)

    cnt = _cnt_call(row32, vsrc_pad).reshape(NC, 2, NPAD)
    inv_row_flat = 1.0 / jnp.clip(cnt[0, 0] + cnt[1, 0], 1.0)
    inv_v2f = 1.0 / jnp.clip(cnt[0, 1, :n] + cnt[1, 1, :n], 1.0)[:, None]
    face0 = face[0].astype(jnp.int32)
    face1 = face[1].astype(jnp.int32)
    face2 = face[2].astype(jnp.int32)

    def area_of(c4):
        fa = _fa_call(c4, face0, face1, face2)
        asum = _a2_call(fa, fidx_pad, vsrc_pad).reshape(NC, NPAD)
        return (asum[0, :n] + asum[1, :n])[:, None] * inv_v2f

    area = area_of(coord4)

    wf, bf = params['feat']
    dims = [(32, 64, 32), (64, 128, 64), (128, 256, 128)]

    def e1_split(i, fi):
        w, b = params['c%d_e1' % i]
        hid = w.shape[1]
        wa = jnp.pad(w[:fi], ((0, 0), (0, 128 - hid)))
        wb = jnp.pad(w[fi:2 * fi], ((0, 0), (0, 128 - hid)))
        return wa, wb, w[2 * fi:2 * fi + 3], b[None, :]

    wa0, wb0, wrwd0, b1e0 = e1_split(0, 32)
    x, xa, xb = _feat_call(area, hks, wf, bf[None, :], wa0, wb0)

    for i, (fi, fo, hid) in enumerate(dims):
        _, _, wrwd, b1e = e1_split(i, fi)
        w2, b2 = params['c%d_e2' % i]
        wx1, bx1 = params['c%d_x1' % i]
        wx2, bx2 = params['c%d_x2' % i]
        wn1, bn1 = params['c%d_n1' % i]
        wn2, bn2 = params['c%d_n2' % i]
        w1x, w1m, w1a = wn1[:fi], wn1[fi:fi + hid], wn1[fi + hid:fi + hid + 1]

        pre, d4 = _eg_call(xa, xb, coord4, row_g, col_g)
        mij, tw = _edge_call(pre, d4.reshape(EPAD, 4), weight,
                             di_angles, wrwd, b1e, w2, b2[None, :],
                             wx1, bx1[None, :], wx2, bx2[None, :])
        magg, coord4 = _es_call(hid <= 124, mij, tw.reshape(-1), d4, row_s,
                                coord4, inv_row_flat)
        area_i = area_of(coord4)

        if i < 2:
            fi2 = dims[i + 1][0]
            wa, wb, _, _ = e1_split(i + 1, fi2)
            hid_next = 128
            x, xa, xb = _node_call(
                False, x, magg, area_i, w1x, w1m, w1a, bn1[None, :],
                wn2, bn2[None, :], wa, wb,
                jnp.zeros((1, hid_next), jnp.float32),
                jnp.zeros((1, hid_next), jnp.float32), fo, hid_next)
        else:
            wl1, bl1 = params['lin1']
            wl2, bl2 = params['lin2']
            x, _, _ = _node_call(
                True, x, magg, area_i, w1x, w1m, w1a, bn1[None, :],
                wn2, bn2[None, :], wl1, wl2,
                bl1[None, :], bl2[None, :], 16, 8)
    return x
